# Initial kernel scaffold; baseline (speedup 1.0000x reference)
import jax
import jax.numpy as jnp
from jax.experimental import pallas as pl


def kernel(features, edge_index, W1, b1, gamma1, beta1, W2, b2, gamma2, beta2, W3, b3, gamma3, beta3, Wc, bc):
    raise NotImplementedError("write your pallas kernel here")


# speedup vs baseline: 5.9234x; 5.9234x over previous
import functools

import jax
import jax.numpy as jnp
from jax import lax
from jax.experimental import pallas as pl
from jax.experimental.pallas import tpu as pltpu
from jax.experimental.pallas import tpu_sc as plsc

N = 10000
E = 320000
D = 128
C = 64
EPS = 1e-5

NC = 2
NS = 16
NW = NC * NS
EW = E // NW
K = 80
NCH = EW // K
KA = 80
NCHA = EW // KA
GA = 25
DH = 64
NP = 10240
RPS = NP // NS
DEGW = 16


def _sc_mesh():
    return plsc.VectorSubcoreMesh(
        core_axis_name="c", subcore_axis_name="s", num_cores=NC, num_subcores=NS
    )


@functools.cache
def _make_deg_kernel():
    return functools.partial(
        pl.kernel,
        out_type=jax.ShapeDtypeStruct((NC, 2, NP, DEGW), jnp.float32),
        mesh=_sc_mesh(),
        scratch_types=[
            pltpu.VMEM((NCH, K), jnp.int32),
            pltpu.VMEM((NCH, K), jnp.int32),
            pltpu.VMEM((K, DEGW), jnp.float32),
            pltpu.VMEM((K, DEGW), jnp.float32),
            pltpu.VMEM_SHARED((NP, DEGW), jnp.float32),
            pltpu.VMEM_SHARED((NP, DEGW), jnp.float32),
        ],
    )(_deg_body)


def _deg_body(src_hbm, dst_hbm, out_hbm, sidx, didx, ones_v, zer_v, dsrc, ddst):
    cid = lax.axis_index("c")
    sid = lax.axis_index("s")
    wid = cid * NS + sid

    one16 = jnp.ones((16,), jnp.float32)
    zero16 = jnp.zeros((16,), jnp.float32)

    def fill(r, _):
        ones_v[r, pl.ds(0, 16)] = one16
        zer_v[r, pl.ds(0, 16)] = zero16
        return 0

    lax.fori_loop(0, K, fill, 0)

    base = sid * RPS
    for t in range(RPS // K):
        pltpu.sync_copy(zer_v, dsrc.at[pl.ds(base + t * K, K)])
        pltpu.sync_copy(zer_v, ddst.at[pl.ds(base + t * K, K)])

    pltpu.sync_copy(src_hbm.at[wid], sidx)
    pltpu.sync_copy(dst_hbm.at[wid], didx)
    plsc.subcore_barrier()

    for j in range(NCH):
        pltpu.sync_copy(ones_v, dsrc.at[sidx.at[j]], add=True)
        pltpu.sync_copy(ones_v, ddst.at[didx.at[j]], add=True)
    plsc.subcore_barrier()

    for t in range(RPS // K):
        lo = base + t * K
        pltpu.sync_copy(dsrc.at[pl.ds(lo, K)], ones_v)
        pltpu.sync_copy(ones_v, out_hbm.at[cid, 0, pl.ds(lo, K)])
    for t in range(RPS // K):
        lo = base + t * K
        pltpu.sync_copy(ddst.at[pl.ds(lo, K)], ones_v)
        pltpu.sync_copy(ones_v, out_hbm.at[cid, 1, pl.ds(lo, K)])


@functools.cache
def _make_agg_kernel():
    return functools.partial(
        pl.kernel,
        out_type=jax.ShapeDtypeStruct((NC, NP, D), jnp.float32),
        mesh=_sc_mesh(),
        scratch_types=[
            pltpu.VMEM((KA,), jnp.int32),
            pltpu.VMEM((KA,), jnp.int32),
            pltpu.VMEM((KA,), jnp.int32),
            pltpu.VMEM((KA,), jnp.int32),
            pltpu.VMEM((KA, D), jnp.float32),
            pltpu.VMEM((KA, D), jnp.float32),
            pltpu.SemaphoreType.DMA,
            pltpu.SemaphoreType.DMA,
            pltpu.SemaphoreType.DMA,
            pltpu.SemaphoreType.DMA,
            pltpu.VMEM_SHARED((NP, D), jnp.float32),
        ],
    )(_agg_body)


def _agg_body(h_hbm, src_hbm, dst_hbm, out_hbm,
              sidx0, sidx1, didx0, didx1, rows0, rows1,
              semi0, semi1, semg0, semg1, acc):
    cid = lax.axis_index("c")
    sid = lax.axis_index("s")
    wid = cid * NS + sid
    eoff = wid * EW

    zero16 = jnp.zeros((16,), jnp.float32)

    def zbody(t, _):
        r = t // (D // 16)
        c = t % (D // 16)
        rows0[r, pl.ds(c * 16, 16)] = zero16
        return 0

    lax.fori_loop(0, KA * (D // 16), zbody, 0)

    base = sid * RPS
    for t in range(RPS // KA):
        pltpu.sync_copy(rows0, acc.at[pl.ds(base + t * KA, KA)])
    plsc.subcore_barrier()

    sidx = (sidx0, sidx1)
    didx = (didx0, didx1)
    rows = (rows0, rows1)
    semi = (semi0, semi1)
    semg = (semg0, semg1)

    def idx_start(j):
        p = j % 2
        return (
            pltpu.async_copy(src_hbm.at[pl.ds(eoff + j * KA, KA)], sidx[p], semi[p]),
            pltpu.async_copy(dst_hbm.at[pl.ds(eoff + j * KA, KA)], didx[p], semi[p]),
        )

    def gather_start(j):
        p = j % 2
        return pltpu.async_copy(h_hbm.at[sidx[p]], rows[p], semg[p])

    def scat(j):
        p = j % 2
        pltpu.sync_copy(rows[p], acc.at[didx[p]], add=True)

    idesc = [None] * NCHA
    gdesc = [None] * NCHA
    idesc[0] = idx_start(0)
    idesc[1] = idx_start(1)
    idesc[0][0].wait()
    idesc[0][1].wait()
    gdesc[0] = gather_start(0)
    for j in range(NCHA):
        if j + 1 < NCHA:
            idesc[j + 1][0].wait()
            idesc[j + 1][1].wait()
            gdesc[j + 1] = gather_start(j + 1)
        gdesc[j].wait()
        scat(j)
        if j + 2 < NCHA:
            idesc[j + 2] = idx_start(j + 2)

    plsc.subcore_barrier()
    for t in range(RPS // KA):
        lo = base + t * KA
        pltpu.sync_copy(acc.at[pl.ds(lo, KA)], rows0)
        pltpu.sync_copy(rows0, out_hbm.at[cid, pl.ds(lo, KA)])


def _norm_body(dsrc_ref, ddst_ref, ns_ref, nd_ref):
    ds_ = dsrc_ref[0][:, 0:1] + dsrc_ref[1][:, 0:1]
    dd_ = ddst_ref[0][:, 0:1] + ddst_ref[1][:, 0:1]
    ns_ref[...] = jnp.where(ds_ > 0, lax.rsqrt(jnp.maximum(ds_, 1.0)), 0.0)
    nd_ref[...] = jnp.where(dd_ > 0, lax.rsqrt(jnp.maximum(dd_, 1.0)), 0.0)


def _norms(deg_src, deg_dst):
    return pl.pallas_call(
        _norm_body,
        out_shape=(
            jax.ShapeDtypeStruct((NP, 1), jnp.float32),
            jax.ShapeDtypeStruct((NP, 1), jnp.float32),
        ),
    )(deg_src, deg_dst)


def _lin_body(x_ref, n_ref, w_ref, o_ref):
    o_ref[...] = jnp.dot(
        x_ref[...] * n_ref[...], w_ref[...], preferred_element_type=jnp.float32
    )


def _lin(x, norm_src, w):
    return pl.pallas_call(
        _lin_body, out_shape=jax.ShapeDtypeStruct((NP, D), jnp.float32)
    )(x, norm_src, w)


def _bn_relu(y, g_ref, be_ref):
    yv = y[:N]
    mu = jnp.mean(yv, axis=0, keepdims=True)
    var = jnp.mean((yv - mu) ** 2, axis=0, keepdims=True)
    h = (y - mu) * lax.rsqrt(var + EPS) * g_ref[...] + be_ref[...]
    return jnp.maximum(h, 0.0)


def _post_body(p_ref, nd_ref, b_ref, g_ref, be_ref, o_ref):
    y = (p_ref[0] + p_ref[1]) * nd_ref[...] + b_ref[...]
    o_ref[...] = _bn_relu(y, g_ref, be_ref)


def _post(p, norm_dst, b, g, be):
    return pl.pallas_call(
        _post_body, out_shape=jax.ShapeDtypeStruct((NP, D), jnp.float32)
    )(p, norm_dst, b, g, be)


def _post3_body(p_ref, nd_ref, b_ref, g_ref, be_ref, res_ref, wc_ref, bc_ref, o_ref):
    y = (p_ref[0] + p_ref[1]) * nd_ref[...] + b_ref[...] + res_ref[...]
    h = _bn_relu(y, g_ref, be_ref)
    o_ref[...] = jnp.dot(h, wc_ref[...], preferred_element_type=jnp.float32) + bc_ref[...]


def _post3(p, norm_dst, b, g, be, res, wc, bc):
    return pl.pallas_call(
        _post3_body, out_shape=jax.ShapeDtypeStruct((NP, C), jnp.float32)
    )(p, norm_dst, b, g, be, res, wc, bc)


@jax.jit
def kernel(features, edge_index, W1, b1, gamma1, beta1, W2, b2, gamma2, beta2,
           W3, b3, gamma3, beta3, Wc, bc):
    srcf = edge_index[0]
    dstf = edge_index[1]
    xp = jnp.pad(features, ((0, NP - N), (0, 0)))

    agg = _make_agg_kernel()
    ones_t = jnp.ones((NP, D), jnp.float32)
    deg_dst = agg(ones_t, srcf, dstf)
    deg_src = agg(ones_t, dstf, srcf)
    norm_src, norm_dst = _norms(deg_src, deg_dst)

    r2 = lambda v: v.reshape(1, -1)

    h1 = _post(agg(_lin(xp, norm_src, W1), srcf, dstf),
               norm_dst, r2(b1), r2(gamma1), r2(beta1))
    h2 = _post(agg(_lin(h1, norm_src, W2), srcf, dstf),
               norm_dst, r2(b2), r2(gamma2), r2(beta2))
    out = _post3(agg(_lin(h2, norm_src, W3), srcf, dstf),
                 norm_dst, r2(b3), r2(gamma3), r2(beta3), h1, Wc, r2(bc))
    return out[:N]


# speedup vs baseline: 7.2450x; 1.2231x over previous
import functools

import jax
import jax.numpy as jnp
from jax import lax
from jax.experimental import pallas as pl
from jax.experimental.pallas import tpu as pltpu
from jax.experimental.pallas import tpu_sc as plsc

N = 10000
E = 320000
D = 128
C = 64
EPS = 1e-5

NC = 2
NS = 16
NW = NC * NS
EW = E // NW
K = 80
NCH = EW // K
KA = 80
NCHA = EW // KA
GA = 25
DH = 64
NP = 10240
RPS = NP // NS
DEGW = 16


def _sc_mesh():
    return plsc.VectorSubcoreMesh(
        core_axis_name="c", subcore_axis_name="s", num_cores=NC, num_subcores=NS
    )


@functools.cache
def _make_deg_kernel():
    return functools.partial(
        pl.kernel,
        out_type=jax.ShapeDtypeStruct((NC, 2, NP, DEGW), jnp.float32),
        mesh=_sc_mesh(),
        scratch_types=[
            pltpu.VMEM((NCH, K), jnp.int32),
            pltpu.VMEM((NCH, K), jnp.int32),
            pltpu.VMEM((K, DEGW), jnp.float32),
            pltpu.VMEM((K, DEGW), jnp.float32),
            pltpu.VMEM_SHARED((NP, DEGW), jnp.float32),
            pltpu.VMEM_SHARED((NP, DEGW), jnp.float32),
        ],
    )(_deg_body)


def _deg_body(src_hbm, dst_hbm, out_hbm, sidx, didx, ones_v, zer_v, dsrc, ddst):
    cid = lax.axis_index("c")
    sid = lax.axis_index("s")
    wid = cid * NS + sid

    one16 = jnp.ones((16,), jnp.float32)
    zero16 = jnp.zeros((16,), jnp.float32)

    def fill(r, _):
        ones_v[r, pl.ds(0, 16)] = one16
        zer_v[r, pl.ds(0, 16)] = zero16
        return 0

    lax.fori_loop(0, K, fill, 0)

    base = sid * RPS
    for t in range(RPS // K):
        pltpu.sync_copy(zer_v, dsrc.at[pl.ds(base + t * K, K)])
        pltpu.sync_copy(zer_v, ddst.at[pl.ds(base + t * K, K)])

    pltpu.sync_copy(src_hbm.at[wid], sidx)
    pltpu.sync_copy(dst_hbm.at[wid], didx)
    plsc.subcore_barrier()

    for j in range(NCH):
        pltpu.sync_copy(ones_v, dsrc.at[sidx.at[j]], add=True)
        pltpu.sync_copy(ones_v, ddst.at[didx.at[j]], add=True)
    plsc.subcore_barrier()

    for t in range(RPS // K):
        lo = base + t * K
        pltpu.sync_copy(dsrc.at[pl.ds(lo, K)], ones_v)
        pltpu.sync_copy(ones_v, out_hbm.at[cid, 0, pl.ds(lo, K)])
    for t in range(RPS // K):
        lo = base + t * K
        pltpu.sync_copy(ddst.at[pl.ds(lo, K)], ones_v)
        pltpu.sync_copy(ones_v, out_hbm.at[cid, 1, pl.ds(lo, K)])


@functools.cache
def _make_agg_kernel():
    return functools.partial(
        pl.kernel,
        out_type=jax.ShapeDtypeStruct((NC, NP, D), jnp.float32),
        mesh=_sc_mesh(),
        scratch_types=[
            pltpu.VMEM((KA,), jnp.int32),
            pltpu.VMEM((KA,), jnp.int32),
            pltpu.VMEM((KA,), jnp.int32),
            pltpu.VMEM((KA,), jnp.int32),
            pltpu.VMEM((KA, D), jnp.float32),
            pltpu.VMEM((KA, D), jnp.float32),
            pltpu.SemaphoreType.DMA,
            pltpu.SemaphoreType.DMA,
            pltpu.SemaphoreType.DMA,
            pltpu.SemaphoreType.DMA,
            pltpu.VMEM_SHARED((NP, D), jnp.float32),
        ],
    )(_agg_body)


def _agg_body(h_hbm, src_hbm, dst_hbm, out_hbm,
              sidx0, sidx1, didx0, didx1, rows0, rows1,
              semi0, semi1, semg0, semg1, acc):
    cid = lax.axis_index("c")
    sid = lax.axis_index("s")
    wid = cid * NS + sid
    eoff = wid * EW

    zero16 = jnp.zeros((16,), jnp.float32)

    def zbody(t, _):
        r = t // (D // 16)
        c = t % (D // 16)
        rows0[r, pl.ds(c * 16, 16)] = zero16
        return 0

    lax.fori_loop(0, KA * (D // 16), zbody, 0)

    base = sid * RPS
    for t in range(RPS // KA):
        pltpu.sync_copy(rows0, acc.at[pl.ds(base + t * KA, KA)])
    plsc.subcore_barrier()

    sidx = (sidx0, sidx1)
    didx = (didx0, didx1)
    rows = (rows0, rows1)
    semi = (semi0, semi1)
    semg = (semg0, semg1)

    def idx_start(j):
        p = j % 2
        return (
            pltpu.async_copy(src_hbm.at[pl.ds(eoff + j * KA, KA)], sidx[p], semi[p]),
            pltpu.async_copy(dst_hbm.at[pl.ds(eoff + j * KA, KA)], didx[p], semi[p]),
        )

    def gather_start(j):
        p = j % 2
        return pltpu.async_copy(h_hbm.at[sidx[p]], rows[p], semg[p])

    def scat(j):
        p = j % 2
        pltpu.sync_copy(rows[p], acc.at[didx[p]], add=True)

    idesc = [None] * NCHA
    gdesc = [None] * NCHA
    idesc[0] = idx_start(0)
    idesc[1] = idx_start(1)
    idesc[0][0].wait()
    idesc[0][1].wait()
    gdesc[0] = gather_start(0)
    for j in range(NCHA):
        if j + 1 < NCHA:
            idesc[j + 1][0].wait()
            idesc[j + 1][1].wait()
            gdesc[j + 1] = gather_start(j + 1)
        gdesc[j].wait()
        scat(j)
        if j + 2 < NCHA:
            idesc[j + 2] = idx_start(j + 2)

    plsc.subcore_barrier()
    for t in range(RPS // KA):
        lo = base + t * KA
        pltpu.sync_copy(acc.at[pl.ds(lo, KA)], rows0)
        pltpu.sync_copy(rows0, out_hbm.at[cid, pl.ds(lo, KA)])


@functools.cache
def _make_deg2_kernel():
    return functools.partial(
        pl.kernel,
        out_type=jax.ShapeDtypeStruct((NC, NP, D), jnp.float32),
        mesh=_sc_mesh(),
        scratch_types=[
            pltpu.VMEM((KA,), jnp.int32),
            pltpu.VMEM((KA,), jnp.int32),
            pltpu.VMEM((KA, D), jnp.float32),
            pltpu.VMEM((KA, D), jnp.float32),
            pltpu.SemaphoreType.DMA,
            pltpu.SemaphoreType.DMA,
            pltpu.VMEM_SHARED((NP, D), jnp.float32),
        ],
    )(_deg2_body)


def _deg2_body(idx_hbm, out_hbm, idx0, idx1, ones_v, zer_v, semi0, semi1, acc):
    cid = lax.axis_index("c")
    sid = lax.axis_index("s")
    wid = cid * NS + sid
    eoff = wid * EW

    zero16 = jnp.zeros((16,), jnp.float32)
    one16 = jnp.ones((16,), jnp.float32)

    def fbody(t, _):
        r = t // (D // 16)
        c = t % (D // 16)
        ones_v[r, pl.ds(c * 16, 16)] = one16
        zer_v[r, pl.ds(c * 16, 16)] = zero16
        return 0

    lax.fori_loop(0, KA * (D // 16), fbody, 0)

    base = sid * RPS
    for t in range(RPS // KA):
        pltpu.sync_copy(zer_v, acc.at[pl.ds(base + t * KA, KA)])
    plsc.subcore_barrier()

    idx = (idx0, idx1)
    semi = (semi0, semi1)

    def idx_start(j):
        p = j % 2
        return pltpu.async_copy(idx_hbm.at[pl.ds(eoff + j * KA, KA)], idx[p], semi[p])

    idesc = [None] * NCHA
    idesc[0] = idx_start(0)
    idesc[1] = idx_start(1)
    for j in range(NCHA):
        idesc[j].wait()
        pltpu.sync_copy(ones_v, acc.at[idx[j % 2]], add=True)
        if j + 2 < NCHA:
            idesc[j + 2] = idx_start(j + 2)

    plsc.subcore_barrier()
    for t in range(RPS // KA):
        lo = base + t * KA
        pltpu.sync_copy(acc.at[pl.ds(lo, KA)], zer_v)
        pltpu.sync_copy(zer_v, out_hbm.at[cid, pl.ds(lo, KA)])


def _norm_body(dsrc_ref, ddst_ref, ns_ref, nd_ref):
    ds_ = dsrc_ref[0][:, 0:1] + dsrc_ref[1][:, 0:1]
    dd_ = ddst_ref[0][:, 0:1] + ddst_ref[1][:, 0:1]
    ns_ref[...] = jnp.where(ds_ > 0, lax.rsqrt(jnp.maximum(ds_, 1.0)), 0.0)
    nd_ref[...] = jnp.where(dd_ > 0, lax.rsqrt(jnp.maximum(dd_, 1.0)), 0.0)


def _norms(deg_src, deg_dst):
    return pl.pallas_call(
        _norm_body,
        out_shape=(
            jax.ShapeDtypeStruct((NP, 1), jnp.float32),
            jax.ShapeDtypeStruct((NP, 1), jnp.float32),
        ),
    )(deg_src, deg_dst)


def _lin_body(x_ref, n_ref, w_ref, o_ref):
    o_ref[...] = jnp.dot(
        x_ref[...] * n_ref[...], w_ref[...], preferred_element_type=jnp.float32
    )


def _lin(x, norm_src, w):
    return pl.pallas_call(
        _lin_body, out_shape=jax.ShapeDtypeStruct((NP, D), jnp.float32)
    )(x, norm_src, w)


def _bn_relu(y, g_ref, be_ref):
    yv = y[:N]
    mu = jnp.mean(yv, axis=0, keepdims=True)
    var = jnp.mean((yv - mu) ** 2, axis=0, keepdims=True)
    h = (y - mu) * lax.rsqrt(var + EPS) * g_ref[...] + be_ref[...]
    return jnp.maximum(h, 0.0)


def _post_body(p_ref, nd_ref, b_ref, g_ref, be_ref, o_ref):
    y = (p_ref[0] + p_ref[1]) * nd_ref[...] + b_ref[...]
    o_ref[...] = _bn_relu(y, g_ref, be_ref)


def _post(p, norm_dst, b, g, be):
    return pl.pallas_call(
        _post_body, out_shape=jax.ShapeDtypeStruct((NP, D), jnp.float32)
    )(p, norm_dst, b, g, be)


def _post3_body(p_ref, nd_ref, b_ref, g_ref, be_ref, res_ref, wc_ref, bc_ref, o_ref):
    y = (p_ref[0] + p_ref[1]) * nd_ref[...] + b_ref[...] + res_ref[...]
    h = _bn_relu(y, g_ref, be_ref)
    o_ref[...] = jnp.dot(h, wc_ref[...], preferred_element_type=jnp.float32) + bc_ref[...]


def _post3(p, norm_dst, b, g, be, res, wc, bc):
    return pl.pallas_call(
        _post3_body, out_shape=jax.ShapeDtypeStruct((NP, C), jnp.float32)
    )(p, norm_dst, b, g, be, res, wc, bc)


@jax.jit
def kernel(features, edge_index, W1, b1, gamma1, beta1, W2, b2, gamma2, beta2,
           W3, b3, gamma3, beta3, Wc, bc):
    srcf = edge_index[0]
    dstf = edge_index[1]
    xp = jnp.pad(features, ((0, NP - N), (0, 0)))

    agg = _make_agg_kernel()
    deg2 = _make_deg2_kernel()
    deg_dst = deg2(dstf)
    deg_src = deg2(srcf)
    norm_src, norm_dst = _norms(deg_src, deg_dst)

    r2 = lambda v: v.reshape(1, -1)

    h1 = _post(agg(_lin(xp, norm_src, W1), srcf, dstf),
               norm_dst, r2(b1), r2(gamma1), r2(beta1))
    h2 = _post(agg(_lin(h1, norm_src, W2), srcf, dstf),
               norm_dst, r2(b2), r2(gamma2), r2(beta2))
    out = _post3(agg(_lin(h2, norm_src, W3), srcf, dstf),
                 norm_dst, r2(b3), r2(gamma3), r2(beta3), h1, Wc, r2(bc))
    return out[:N]


# speedup vs baseline: 7.6357x; 1.0539x over previous
import functools

import jax
import jax.numpy as jnp
from jax import lax
from jax.experimental import pallas as pl
from jax.experimental.pallas import tpu as pltpu
from jax.experimental.pallas import tpu_sc as plsc

N = 10000
E = 320000
D = 128
C = 64
EPS = 1e-5

NC = 2
NS = 16
NW = NC * NS
EW = E // NW
K = 80
NCH = EW // K
KA = 80
NCHA = EW // KA
GA = 25
DH = 64
NP = 10240
RPS = NP // NS
DEGW = 16


def _sc_mesh():
    return plsc.VectorSubcoreMesh(
        core_axis_name="c", subcore_axis_name="s", num_cores=NC, num_subcores=NS
    )


@functools.cache
def _make_deg_kernel():
    return functools.partial(
        pl.kernel,
        out_type=jax.ShapeDtypeStruct((NC, 2, NP, DEGW), jnp.float32),
        mesh=_sc_mesh(),
        scratch_types=[
            pltpu.VMEM((NCH, K), jnp.int32),
            pltpu.VMEM((NCH, K), jnp.int32),
            pltpu.VMEM((K, DEGW), jnp.float32),
            pltpu.VMEM((K, DEGW), jnp.float32),
            pltpu.VMEM_SHARED((NP, DEGW), jnp.float32),
            pltpu.VMEM_SHARED((NP, DEGW), jnp.float32),
        ],
    )(_deg_body)


def _deg_body(src_hbm, dst_hbm, out_hbm, sidx, didx, ones_v, zer_v, dsrc, ddst):
    cid = lax.axis_index("c")
    sid = lax.axis_index("s")
    wid = cid * NS + sid

    one16 = jnp.ones((16,), jnp.float32)
    zero16 = jnp.zeros((16,), jnp.float32)

    def fill(r, _):
        ones_v[r, pl.ds(0, 16)] = one16
        zer_v[r, pl.ds(0, 16)] = zero16
        return 0

    lax.fori_loop(0, K, fill, 0)

    base = sid * RPS
    for t in range(RPS // K):
        pltpu.sync_copy(zer_v, dsrc.at[pl.ds(base + t * K, K)])
        pltpu.sync_copy(zer_v, ddst.at[pl.ds(base + t * K, K)])

    pltpu.sync_copy(src_hbm.at[wid], sidx)
    pltpu.sync_copy(dst_hbm.at[wid], didx)
    plsc.subcore_barrier()

    for j in range(NCH):
        pltpu.sync_copy(ones_v, dsrc.at[sidx.at[j]], add=True)
        pltpu.sync_copy(ones_v, ddst.at[didx.at[j]], add=True)
    plsc.subcore_barrier()

    for t in range(RPS // K):
        lo = base + t * K
        pltpu.sync_copy(dsrc.at[pl.ds(lo, K)], ones_v)
        pltpu.sync_copy(ones_v, out_hbm.at[cid, 0, pl.ds(lo, K)])
    for t in range(RPS // K):
        lo = base + t * K
        pltpu.sync_copy(ddst.at[pl.ds(lo, K)], ones_v)
        pltpu.sync_copy(ones_v, out_hbm.at[cid, 1, pl.ds(lo, K)])


@functools.cache
def _make_agg_kernel():
    return functools.partial(
        pl.kernel,
        out_type=jax.ShapeDtypeStruct((NC, NP, D), jnp.float32),
        mesh=_sc_mesh(),
        scratch_types=[
            pltpu.VMEM((KA,), jnp.int32),
            pltpu.VMEM((KA,), jnp.int32),
            pltpu.VMEM((KA,), jnp.int32),
            pltpu.VMEM((KA,), jnp.int32),
            pltpu.VMEM((KA, D), jnp.float32),
            pltpu.VMEM((KA, D), jnp.float32),
            pltpu.SemaphoreType.DMA,
            pltpu.SemaphoreType.DMA,
            pltpu.SemaphoreType.DMA,
            pltpu.SemaphoreType.DMA,
            pltpu.VMEM_SHARED((NP, D), jnp.float32),
        ],
    )(_agg_body)


def _agg_body(h_hbm, src_hbm, dst_hbm, out_hbm,
              sidx0, sidx1, didx0, didx1, rows0, rows1,
              semi0, semi1, semg0, semg1, acc):
    cid = lax.axis_index("c")
    sid = lax.axis_index("s")
    wid = cid * NS + sid
    eoff = wid * EW

    zero16 = jnp.zeros((16,), jnp.float32)

    def zbody(t, _):
        r = t // (D // 16)
        c = t % (D // 16)
        rows0[r, pl.ds(c * 16, 16)] = zero16
        return 0

    lax.fori_loop(0, KA * (D // 16), zbody, 0)

    base = sid * RPS
    for t in range(RPS // KA):
        pltpu.sync_copy(rows0, acc.at[pl.ds(base + t * KA, KA)])
    plsc.subcore_barrier()

    sidx = (sidx0, sidx1)
    didx = (didx0, didx1)
    rows = (rows0, rows1)
    semi = (semi0, semi1)
    semg = (semg0, semg1)

    def idx_start(j):
        p = j % 2
        return (
            pltpu.async_copy(src_hbm.at[pl.ds(eoff + j * KA, KA)], sidx[p], semi[p]),
            pltpu.async_copy(dst_hbm.at[pl.ds(eoff + j * KA, KA)], didx[p], semi[p]),
        )

    def gather_start(j):
        p = j % 2
        return pltpu.async_copy(h_hbm.at[sidx[p]], rows[p], semg[p])

    def scat(j):
        p = j % 2
        pltpu.sync_copy(rows[p], acc.at[didx[p]], add=True)

    idesc = [None] * NCHA
    gdesc = [None] * NCHA
    idesc[0] = idx_start(0)
    idesc[1] = idx_start(1)
    idesc[0][0].wait()
    idesc[0][1].wait()
    gdesc[0] = gather_start(0)
    for j in range(NCHA):
        if j + 1 < NCHA:
            idesc[j + 1][0].wait()
            idesc[j + 1][1].wait()
            gdesc[j + 1] = gather_start(j + 1)
        gdesc[j].wait()
        scat(j)
        if j + 2 < NCHA:
            idesc[j + 2] = idx_start(j + 2)

    plsc.subcore_barrier()
    for t in range(RPS // KA):
        lo = base + t * KA
        pltpu.sync_copy(acc.at[pl.ds(lo, KA)], rows0)
        pltpu.sync_copy(rows0, out_hbm.at[cid, pl.ds(lo, KA)])


@functools.cache
def _make_deg2_kernel():
    return functools.partial(
        pl.kernel,
        out_type=jax.ShapeDtypeStruct((NC, NP, D), jnp.float32),
        mesh=_sc_mesh(),
        scratch_types=[
            pltpu.VMEM((KA,), jnp.int32),
            pltpu.VMEM((KA,), jnp.int32),
            pltpu.VMEM((KA, D), jnp.float32),
            pltpu.VMEM((KA, D), jnp.float32),
            pltpu.SemaphoreType.DMA,
            pltpu.SemaphoreType.DMA,
            pltpu.VMEM_SHARED((NP, D), jnp.float32),
        ],
    )(_deg2_body)


def _deg2_body(idx_hbm, out_hbm, idx0, idx1, ones_v, zer_v, semi0, semi1, acc):
    cid = lax.axis_index("c")
    sid = lax.axis_index("s")
    wid = cid * NS + sid
    eoff = wid * EW

    zero16 = jnp.zeros((16,), jnp.float32)
    one16 = jnp.ones((16,), jnp.float32)

    def fbody(t, _):
        r = t // (D // 16)
        c = t % (D // 16)
        ones_v[r, pl.ds(c * 16, 16)] = one16
        zer_v[r, pl.ds(c * 16, 16)] = zero16
        return 0

    lax.fori_loop(0, KA * (D // 16), fbody, 0)

    base = sid * RPS
    for t in range(RPS // KA):
        pltpu.sync_copy(zer_v, acc.at[pl.ds(base + t * KA, KA)])
    plsc.subcore_barrier()

    idx = (idx0, idx1)
    semi = (semi0, semi1)

    def idx_start(j):
        p = j % 2
        return pltpu.async_copy(idx_hbm.at[pl.ds(eoff + j * KA, KA)], idx[p], semi[p])

    idesc = [None] * NCHA
    idesc[0] = idx_start(0)
    idesc[1] = idx_start(1)
    for j in range(NCHA):
        idesc[j].wait()
        pltpu.sync_copy(ones_v, acc.at[idx[j % 2]], add=True)
        if j + 2 < NCHA:
            idesc[j + 2] = idx_start(j + 2)

    plsc.subcore_barrier()
    for t in range(RPS // KA):
        lo = base + t * KA
        pltpu.sync_copy(acc.at[pl.ds(lo, KA)], zer_v)
        pltpu.sync_copy(zer_v, out_hbm.at[cid, pl.ds(lo, KA)])


@functools.cache
def _make_degb_kernel():
    return functools.partial(
        pl.kernel,
        out_type=jax.ShapeDtypeStruct((NC, NP, D), jnp.float32),
        mesh=_sc_mesh(),
        scratch_types=[
            pltpu.VMEM((KA,), jnp.int32),
            pltpu.VMEM((KA,), jnp.int32),
            pltpu.VMEM((KA,), jnp.int32),
            pltpu.VMEM((KA,), jnp.int32),
            pltpu.VMEM((KA, D), jnp.float32),
            pltpu.VMEM((KA, D), jnp.float32),
            pltpu.SemaphoreType.DMA,
            pltpu.SemaphoreType.DMA,
            pltpu.VMEM_SHARED((NP, D), jnp.float32),
        ],
    )(_degb_body)


def _degb_body(src_hbm, dst_hbm, out_hbm, sidx0, sidx1, didx0, didx1,
               ones_lo, ones_hi, semi0, semi1, acc):
    cid = lax.axis_index("c")
    sid = lax.axis_index("s")
    wid = cid * NS + sid
    eoff = wid * EW

    zero16 = jnp.zeros((16,), jnp.float32)
    one16 = jnp.ones((16,), jnp.float32)

    def f1(t, _):
        r = t // (D // 16)
        c = t % (D // 16)
        ones_lo[r, pl.ds(c * 16, 16)] = zero16
        ones_hi[r, pl.ds(c * 16, 16)] = zero16
        return 0

    lax.fori_loop(0, KA * (D // 16), f1, 0)

    base = sid * RPS
    for t in range(RPS // KA):
        pltpu.sync_copy(ones_lo, acc.at[pl.ds(base + t * KA, KA)])

    def f2(t, _):
        r = t // (D // 32)
        c = t % (D // 32)
        ones_lo[r, pl.ds(c * 16, 16)] = one16
        ones_hi[r, pl.ds(D // 2 + c * 16, 16)] = one16
        return 0

    lax.fori_loop(0, KA * (D // 32), f2, 0)
    plsc.subcore_barrier()

    sidx = (sidx0, sidx1)
    didx = (didx0, didx1)
    semi = (semi0, semi1)

    def idx_start(j):
        p = j % 2
        return (
            pltpu.async_copy(src_hbm.at[pl.ds(eoff + j * KA, KA)], sidx[p], semi[p]),
            pltpu.async_copy(dst_hbm.at[pl.ds(eoff + j * KA, KA)], didx[p], semi[p]),
        )

    idesc = [None] * NCHA
    idesc[0] = idx_start(0)
    idesc[1] = idx_start(1)
    for j in range(NCHA):
        p = j % 2
        idesc[j][0].wait()
        idesc[j][1].wait()
        pltpu.sync_copy(ones_hi, acc.at[sidx[p]], add=True)
        pltpu.sync_copy(ones_lo, acc.at[didx[p]], add=True)
        if j + 2 < NCHA:
            idesc[j + 2] = idx_start(j + 2)

    plsc.subcore_barrier()
    for t in range(RPS // KA):
        lo = base + t * KA
        pltpu.sync_copy(acc.at[pl.ds(lo, KA)], ones_lo)
        pltpu.sync_copy(ones_lo, out_hbm.at[cid, pl.ds(lo, KA)])


def _lin1_body(x_ref, deg_ref, w_ref, o_ref, ns_ref, nd_ref):
    ds_ = deg_ref[0][:, D // 2:D // 2 + 1] + deg_ref[1][:, D // 2:D // 2 + 1]
    dd_ = deg_ref[0][:, 0:1] + deg_ref[1][:, 0:1]
    ns = jnp.where(ds_ > 0, lax.rsqrt(jnp.maximum(ds_, 1.0)), 0.0)
    nd = jnp.where(dd_ > 0, lax.rsqrt(jnp.maximum(dd_, 1.0)), 0.0)
    ns_ref[...] = ns
    nd_ref[...] = nd
    o_ref[...] = jnp.dot(x_ref[...] * ns, w_ref[...], preferred_element_type=jnp.float32)


def _lin1(x, degt, w):
    return pl.pallas_call(
        _lin1_body,
        out_shape=(
            jax.ShapeDtypeStruct((NP, D), jnp.float32),
            jax.ShapeDtypeStruct((NP, 1), jnp.float32),
            jax.ShapeDtypeStruct((NP, 1), jnp.float32),
        ),
    )(x, degt, w)


def _postlin_body(p_ref, nd_ref, b_ref, g_ref, be_ref, ns_ref, w_ref, h_ref, o_ref):
    y = (p_ref[0] + p_ref[1]) * nd_ref[...] + b_ref[...]
    h = _bn_relu(y, g_ref, be_ref)
    h_ref[...] = h
    o_ref[...] = jnp.dot(h * ns_ref[...], w_ref[...], preferred_element_type=jnp.float32)


def _postlin(p, norm_dst, b, g, be, norm_src, w):
    return pl.pallas_call(
        _postlin_body,
        out_shape=(
            jax.ShapeDtypeStruct((NP, D), jnp.float32),
            jax.ShapeDtypeStruct((NP, D), jnp.float32),
        ),
    )(p, norm_dst, b, g, be, norm_src, w)


def _bn_relu(y, g_ref, be_ref):
    yv = y[:N]
    mu = jnp.mean(yv, axis=0, keepdims=True)
    var = jnp.mean((yv - mu) ** 2, axis=0, keepdims=True)
    h = (y - mu) * lax.rsqrt(var + EPS) * g_ref[...] + be_ref[...]
    return jnp.maximum(h, 0.0)


def _post_body(p_ref, nd_ref, b_ref, g_ref, be_ref, o_ref):
    y = (p_ref[0] + p_ref[1]) * nd_ref[...] + b_ref[...]
    o_ref[...] = _bn_relu(y, g_ref, be_ref)


def _post(p, norm_dst, b, g, be):
    return pl.pallas_call(
        _post_body, out_shape=jax.ShapeDtypeStruct((NP, D), jnp.float32)
    )(p, norm_dst, b, g, be)


def _post3_body(p_ref, nd_ref, b_ref, g_ref, be_ref, res_ref, wc_ref, bc_ref, o_ref):
    y = (p_ref[0] + p_ref[1]) * nd_ref[...] + b_ref[...] + res_ref[...]
    h = _bn_relu(y, g_ref, be_ref)
    o_ref[...] = jnp.dot(h, wc_ref[...], preferred_element_type=jnp.float32) + bc_ref[...]


def _post3(p, norm_dst, b, g, be, res, wc, bc):
    return pl.pallas_call(
        _post3_body, out_shape=jax.ShapeDtypeStruct((NP, C), jnp.float32)
    )(p, norm_dst, b, g, be, res, wc, bc)


@jax.jit
def kernel(features, edge_index, W1, b1, gamma1, beta1, W2, b2, gamma2, beta2,
           W3, b3, gamma3, beta3, Wc, bc):
    srcf = edge_index[0]
    dstf = edge_index[1]
    xp = jnp.pad(features, ((0, NP - N), (0, 0)))

    agg = _make_agg_kernel()
    degt = _make_degb_kernel()(srcf, dstf)

    r2 = lambda v: v.reshape(1, -1)

    H1, norm_src, norm_dst = _lin1(xp, degt, W1)
    h1, H2 = _postlin(agg(H1, srcf, dstf), norm_dst,
                      r2(b1), r2(gamma1), r2(beta1), norm_src, W2)
    h2, H3 = _postlin(agg(H2, srcf, dstf), norm_dst,
                      r2(b2), r2(gamma2), r2(beta2), norm_src, W3)
    out = _post3(agg(H3, srcf, dstf), norm_dst,
                 r2(b3), r2(gamma3), r2(beta3), h1, Wc, r2(bc))
    return out[:N]


# speedup vs baseline: 8.5205x; 1.1159x over previous
import functools

import jax
import jax.numpy as jnp
from jax import lax
from jax.experimental import pallas as pl
from jax.experimental.pallas import tpu as pltpu
from jax.experimental.pallas import tpu_sc as plsc

N = 10000
E = 320000
D = 128
C = 64
EPS = 1e-5

NC = 2
NS = 16
NW = NC * NS
EW = E // NW
K = 80
NCH = EW // K
KA = 80
NCHA = EW // KA
GA = 25
DH = 64
NP = 10240
RPS = NP // NS
DEGW = 16


def _sc_mesh():
    return plsc.VectorSubcoreMesh(
        core_axis_name="c", subcore_axis_name="s", num_cores=NC, num_subcores=NS
    )


@functools.cache
def _make_deg_kernel():
    return functools.partial(
        pl.kernel,
        out_type=jax.ShapeDtypeStruct((NC, 2, NP, DEGW), jnp.float32),
        mesh=_sc_mesh(),
        scratch_types=[
            pltpu.VMEM((NCH, K), jnp.int32),
            pltpu.VMEM((NCH, K), jnp.int32),
            pltpu.VMEM((K, DEGW), jnp.float32),
            pltpu.VMEM((K, DEGW), jnp.float32),
            pltpu.VMEM_SHARED((NP, DEGW), jnp.float32),
            pltpu.VMEM_SHARED((NP, DEGW), jnp.float32),
        ],
    )(_deg_body)


def _deg_body(src_hbm, dst_hbm, out_hbm, sidx, didx, ones_v, zer_v, dsrc, ddst):
    cid = lax.axis_index("c")
    sid = lax.axis_index("s")
    wid = cid * NS + sid

    one16 = jnp.ones((16,), jnp.float32)
    zero16 = jnp.zeros((16,), jnp.float32)

    def fill(r, _):
        ones_v[r, pl.ds(0, 16)] = one16
        zer_v[r, pl.ds(0, 16)] = zero16
        return 0

    lax.fori_loop(0, K, fill, 0)

    base = sid * RPS
    for t in range(RPS // K):
        pltpu.sync_copy(zer_v, dsrc.at[pl.ds(base + t * K, K)])
        pltpu.sync_copy(zer_v, ddst.at[pl.ds(base + t * K, K)])

    pltpu.sync_copy(src_hbm.at[wid], sidx)
    pltpu.sync_copy(dst_hbm.at[wid], didx)
    plsc.subcore_barrier()

    for j in range(NCH):
        pltpu.sync_copy(ones_v, dsrc.at[sidx.at[j]], add=True)
        pltpu.sync_copy(ones_v, ddst.at[didx.at[j]], add=True)
    plsc.subcore_barrier()

    for t in range(RPS // K):
        lo = base + t * K
        pltpu.sync_copy(dsrc.at[pl.ds(lo, K)], ones_v)
        pltpu.sync_copy(ones_v, out_hbm.at[cid, 0, pl.ds(lo, K)])
    for t in range(RPS // K):
        lo = base + t * K
        pltpu.sync_copy(ddst.at[pl.ds(lo, K)], ones_v)
        pltpu.sync_copy(ones_v, out_hbm.at[cid, 1, pl.ds(lo, K)])


@functools.cache
def _make_agg_kernel():
    return functools.partial(
        pl.kernel,
        out_type=jax.ShapeDtypeStruct((NC, NP, D), jnp.float32),
        mesh=_sc_mesh(),
        scratch_types=[
            pltpu.VMEM((KA,), jnp.int32),
            pltpu.VMEM((KA,), jnp.int32),
            pltpu.VMEM((KA,), jnp.int32),
            pltpu.VMEM((KA,), jnp.int32),
            pltpu.VMEM((KA,), jnp.int32),
            pltpu.VMEM((KA,), jnp.int32),
            pltpu.VMEM((KA, D), jnp.float32),
            pltpu.VMEM((KA, D), jnp.float32),
            pltpu.SemaphoreType.DMA,
            pltpu.SemaphoreType.DMA,
            pltpu.SemaphoreType.DMA,
            pltpu.SemaphoreType.DMA,
            pltpu.SemaphoreType.DMA,
            pltpu.SemaphoreType.DMA,
            pltpu.SemaphoreType.DMA,
            pltpu.VMEM_SHARED((NP, D), jnp.float32),
        ],
    )(_agg_body)


def _agg_body(h_hbm, src_hbm, dst_hbm, out_hbm,
              sidx0, sidx1, sidx2, didx0, didx1, didx2, rows0, rows1,
              semi0, semi1, semi2, semg0, semg1, sems0, sems1, acc):
    cid = lax.axis_index("c")
    sid = lax.axis_index("s")
    wid = cid * NS + sid
    eoff = wid * EW

    zero16 = jnp.zeros((16,), jnp.float32)

    def zbody(t, _):
        r = t // (D // 16)
        c = t % (D // 16)
        rows0[r, pl.ds(c * 16, 16)] = zero16
        return 0

    lax.fori_loop(0, KA * (D // 16), zbody, 0)

    base = sid * RPS
    for t in range(RPS // KA):
        pltpu.sync_copy(rows0, acc.at[pl.ds(base + t * KA, KA)])
    plsc.subcore_barrier()

    sidx = (sidx0, sidx1, sidx2)
    didx = (didx0, didx1, didx2)
    rows = (rows0, rows1)
    semi = (semi0, semi1, semi2)
    semg = (semg0, semg1)
    sems = (sems0, sems1)

    def idx_start(j):
        p = j % 3
        return (
            pltpu.async_copy(src_hbm.at[pl.ds(eoff + j * KA, KA)], sidx[p], semi[p]),
            pltpu.async_copy(dst_hbm.at[pl.ds(eoff + j * KA, KA)], didx[p], semi[p]),
        )

    def gather_start(j):
        return pltpu.async_copy(h_hbm.at[sidx[j % 3]], rows[j % 2], semg[j % 2])

    def scat_start(j):
        return pltpu.async_copy(rows[j % 2], acc.at[didx[j % 3]], sems[j % 2], add=True)

    idesc = [None] * NCHA
    gdesc = [None] * NCHA
    sdesc = [None] * NCHA
    idesc[0] = idx_start(0)
    idesc[1] = idx_start(1)
    idesc[0][0].wait()
    idesc[0][1].wait()
    gdesc[0] = gather_start(0)
    for j in range(NCHA):
        if j >= 1:
            sdesc[j - 1].wait()
        if j + 1 < NCHA:
            idesc[j + 1][0].wait()
            idesc[j + 1][1].wait()
            gdesc[j + 1] = gather_start(j + 1)
        gdesc[j].wait()
        sdesc[j] = scat_start(j)
        if j + 2 < NCHA:
            idesc[j + 2] = idx_start(j + 2)
    sdesc[NCHA - 1].wait()

    plsc.subcore_barrier()
    for t in range(RPS // KA):
        lo = base + t * KA
        pltpu.sync_copy(acc.at[pl.ds(lo, KA)], rows0)
        pltpu.sync_copy(rows0, out_hbm.at[cid, pl.ds(lo, KA)])


@functools.cache
def _make_deg2_kernel():
    return functools.partial(
        pl.kernel,
        out_type=jax.ShapeDtypeStruct((NC, NP, D), jnp.float32),
        mesh=_sc_mesh(),
        scratch_types=[
            pltpu.VMEM((KA,), jnp.int32),
            pltpu.VMEM((KA,), jnp.int32),
            pltpu.VMEM((KA, D), jnp.float32),
            pltpu.VMEM((KA, D), jnp.float32),
            pltpu.SemaphoreType.DMA,
            pltpu.SemaphoreType.DMA,
            pltpu.VMEM_SHARED((NP, D), jnp.float32),
        ],
    )(_deg2_body)


def _deg2_body(idx_hbm, out_hbm, idx0, idx1, ones_v, zer_v, semi0, semi1, acc):
    cid = lax.axis_index("c")
    sid = lax.axis_index("s")
    wid = cid * NS + sid
    eoff = wid * EW

    zero16 = jnp.zeros((16,), jnp.float32)
    one16 = jnp.ones((16,), jnp.float32)

    def fbody(t, _):
        r = t // (D // 16)
        c = t % (D // 16)
        ones_v[r, pl.ds(c * 16, 16)] = one16
        zer_v[r, pl.ds(c * 16, 16)] = zero16
        return 0

    lax.fori_loop(0, KA * (D // 16), fbody, 0)

    base = sid * RPS
    for t in range(RPS // KA):
        pltpu.sync_copy(zer_v, acc.at[pl.ds(base + t * KA, KA)])
    plsc.subcore_barrier()

    idx = (idx0, idx1)
    semi = (semi0, semi1)

    def idx_start(j):
        p = j % 2
        return pltpu.async_copy(idx_hbm.at[pl.ds(eoff + j * KA, KA)], idx[p], semi[p])

    idesc = [None] * NCHA
    idesc[0] = idx_start(0)
    idesc[1] = idx_start(1)
    for j in range(NCHA):
        idesc[j].wait()
        pltpu.sync_copy(ones_v, acc.at[idx[j % 2]], add=True)
        if j + 2 < NCHA:
            idesc[j + 2] = idx_start(j + 2)

    plsc.subcore_barrier()
    for t in range(RPS // KA):
        lo = base + t * KA
        pltpu.sync_copy(acc.at[pl.ds(lo, KA)], zer_v)
        pltpu.sync_copy(zer_v, out_hbm.at[cid, pl.ds(lo, KA)])


@functools.cache
def _make_degb_kernel():
    return functools.partial(
        pl.kernel,
        out_type=jax.ShapeDtypeStruct((NC, NP, D), jnp.float32),
        mesh=_sc_mesh(),
        scratch_types=[
            pltpu.VMEM((KA,), jnp.int32),
            pltpu.VMEM((KA,), jnp.int32),
            pltpu.VMEM((KA,), jnp.int32),
            pltpu.VMEM((KA,), jnp.int32),
            pltpu.VMEM((KA, D), jnp.float32),
            pltpu.VMEM((KA, D), jnp.float32),
            pltpu.SemaphoreType.DMA,
            pltpu.SemaphoreType.DMA,
            pltpu.VMEM_SHARED((NP, D), jnp.float32),
        ],
    )(_degb_body)


def _degb_body(src_hbm, dst_hbm, out_hbm, sidx0, sidx1, didx0, didx1,
               ones_lo, ones_hi, semi0, semi1, acc):
    cid = lax.axis_index("c")
    sid = lax.axis_index("s")
    wid = cid * NS + sid
    eoff = wid * EW

    zero16 = jnp.zeros((16,), jnp.float32)
    one16 = jnp.ones((16,), jnp.float32)

    def f1(t, _):
        r = t // (D // 16)
        c = t % (D // 16)
        ones_lo[r, pl.ds(c * 16, 16)] = zero16
        ones_hi[r, pl.ds(c * 16, 16)] = zero16
        return 0

    lax.fori_loop(0, KA * (D // 16), f1, 0)

    base = sid * RPS
    for t in range(RPS // KA):
        pltpu.sync_copy(ones_lo, acc.at[pl.ds(base + t * KA, KA)])

    def f2(t, _):
        r = t // (D // 32)
        c = t % (D // 32)
        ones_lo[r, pl.ds(c * 16, 16)] = one16
        ones_hi[r, pl.ds(D // 2 + c * 16, 16)] = one16
        return 0

    lax.fori_loop(0, KA * (D // 32), f2, 0)
    plsc.subcore_barrier()

    sidx = (sidx0, sidx1)
    didx = (didx0, didx1)
    semi = (semi0, semi1)

    def idx_start(j):
        p = j % 2
        return (
            pltpu.async_copy(src_hbm.at[pl.ds(eoff + j * KA, KA)], sidx[p], semi[p]),
            pltpu.async_copy(dst_hbm.at[pl.ds(eoff + j * KA, KA)], didx[p], semi[p]),
        )

    idesc = [None] * NCHA
    idesc[0] = idx_start(0)
    idesc[1] = idx_start(1)
    for j in range(NCHA):
        p = j % 2
        idesc[j][0].wait()
        idesc[j][1].wait()
        pltpu.sync_copy(ones_hi, acc.at[sidx[p]], add=True)
        pltpu.sync_copy(ones_lo, acc.at[didx[p]], add=True)
        if j + 2 < NCHA:
            idesc[j + 2] = idx_start(j + 2)

    plsc.subcore_barrier()
    for t in range(RPS // KA):
        lo = base + t * KA
        pltpu.sync_copy(acc.at[pl.ds(lo, KA)], ones_lo)
        pltpu.sync_copy(ones_lo, out_hbm.at[cid, pl.ds(lo, KA)])


def _lin1_body(x_ref, deg_ref, w_ref, o_ref, ns_ref, nd_ref):
    ds_ = deg_ref[0][:, D // 2:D // 2 + 1] + deg_ref[1][:, D // 2:D // 2 + 1]
    dd_ = deg_ref[0][:, 0:1] + deg_ref[1][:, 0:1]
    ns = jnp.where(ds_ > 0, lax.rsqrt(jnp.maximum(ds_, 1.0)), 0.0)
    nd = jnp.where(dd_ > 0, lax.rsqrt(jnp.maximum(dd_, 1.0)), 0.0)
    ns_ref[...] = ns
    nd_ref[...] = nd
    o_ref[...] = jnp.dot(x_ref[...] * ns, w_ref[...], preferred_element_type=jnp.float32)


def _lin1(x, degt, w):
    return pl.pallas_call(
        _lin1_body,
        out_shape=(
            jax.ShapeDtypeStruct((NP, D), jnp.float32),
            jax.ShapeDtypeStruct((NP, 1), jnp.float32),
            jax.ShapeDtypeStruct((NP, 1), jnp.float32),
        ),
    )(x, degt, w)


def _postlin_body(p_ref, nd_ref, b_ref, g_ref, be_ref, ns_ref, w_ref, h_ref, o_ref):
    y = (p_ref[0] + p_ref[1]) * nd_ref[...] + b_ref[...]
    h = _bn_relu(y, g_ref, be_ref)
    h_ref[...] = h
    o_ref[...] = jnp.dot(h * ns_ref[...], w_ref[...], preferred_element_type=jnp.float32)


def _postlin(p, norm_dst, b, g, be, norm_src, w):
    return pl.pallas_call(
        _postlin_body,
        out_shape=(
            jax.ShapeDtypeStruct((NP, D), jnp.float32),
            jax.ShapeDtypeStruct((NP, D), jnp.float32),
        ),
    )(p, norm_dst, b, g, be, norm_src, w)


def _bn_relu(y, g_ref, be_ref):
    yv = y[:N]
    mu = jnp.mean(yv, axis=0, keepdims=True)
    var = jnp.mean((yv - mu) ** 2, axis=0, keepdims=True)
    h = (y - mu) * lax.rsqrt(var + EPS) * g_ref[...] + be_ref[...]
    return jnp.maximum(h, 0.0)


def _post_body(p_ref, nd_ref, b_ref, g_ref, be_ref, o_ref):
    y = (p_ref[0] + p_ref[1]) * nd_ref[...] + b_ref[...]
    o_ref[...] = _bn_relu(y, g_ref, be_ref)


def _post(p, norm_dst, b, g, be):
    return pl.pallas_call(
        _post_body, out_shape=jax.ShapeDtypeStruct((NP, D), jnp.float32)
    )(p, norm_dst, b, g, be)


def _post3_body(p_ref, nd_ref, b_ref, g_ref, be_ref, res_ref, wc_ref, bc_ref, o_ref):
    y = (p_ref[0] + p_ref[1]) * nd_ref[...] + b_ref[...] + res_ref[...]
    h = _bn_relu(y, g_ref, be_ref)
    o_ref[...] = jnp.dot(h, wc_ref[...], preferred_element_type=jnp.float32) + bc_ref[...]


def _post3(p, norm_dst, b, g, be, res, wc, bc):
    return pl.pallas_call(
        _post3_body, out_shape=jax.ShapeDtypeStruct((NP, C), jnp.float32)
    )(p, norm_dst, b, g, be, res, wc, bc)


@jax.jit
def kernel(features, edge_index, W1, b1, gamma1, beta1, W2, b2, gamma2, beta2,
           W3, b3, gamma3, beta3, Wc, bc):
    srcf = edge_index[0]
    dstf = edge_index[1]
    xp = jnp.pad(features, ((0, NP - N), (0, 0)))

    agg = _make_agg_kernel()
    degt = _make_degb_kernel()(srcf, dstf)

    r2 = lambda v: v.reshape(1, -1)

    H1, norm_src, norm_dst = _lin1(xp, degt, W1)
    h1, H2 = _postlin(agg(H1, srcf, dstf), norm_dst,
                      r2(b1), r2(gamma1), r2(beta1), norm_src, W2)
    h2, H3 = _postlin(agg(H2, srcf, dstf), norm_dst,
                      r2(b2), r2(gamma2), r2(beta2), norm_src, W3)
    out = _post3(agg(H3, srcf, dstf), norm_dst,
                 r2(b3), r2(gamma3), r2(beta3), h1, Wc, r2(bc))
    return out[:N]
